# Initial kernel scaffold; baseline (speedup 1.0000x reference)
import functools

import jax
import jax.numpy as jnp
from jax import lax
from jax.experimental import pallas as pl
from jax.experimental.pallas import tpu as pltpu
from jax.experimental.pallas import tpu_sc as plsc

N_NODES = 10000
N_EDGES = 320000
F = 128

NC = 2
NS = 16
NW = NC * NS

CHUNK = 128
K = -(-N_EDGES // (NW * CHUNK))
E_PAD = NW * CHUNK * K
ROWS_PER_TILE = N_NODES // NS
PAD_ROWS = 8

_sc_mesh = plsc.VectorSubcoreMesh(core_axis_name="c", subcore_axis_name="s")


@functools.partial(
    pl.kernel,
    out_type=jax.ShapeDtypeStruct((NC, N_NODES, F), jnp.float32),
    mesh=_sc_mesh,
    scratch_types=[
        pltpu.VMEM_SHARED((N_NODES + PAD_ROWS, F), jnp.float32),
        pltpu.VMEM((CHUNK,), jnp.int32),
        pltpu.VMEM((CHUNK,), jnp.int32),
        pltpu.VMEM((CHUNK, F), jnp.float32),
        pltpu.SemaphoreType.DMA,
    ],
)
def _sc_aggregate(x_hbm, src_hbm, dst_hbm, z_hbm, parts_hbm,
                  agg_s, src_v, dst_v, rows_v, sem):
    c = lax.axis_index("c")
    s = lax.axis_index("s")
    wid = c * NS + s

    pltpu.sync_copy(z_hbm.at[pl.ds(0, ROWS_PER_TILE)],
                    agg_s.at[pl.ds(s * ROWS_PER_TILE, ROWS_PER_TILE)])

    @pl.when(s == 0)
    def _():
        pltpu.sync_copy(z_hbm.at[pl.ds(0, PAD_ROWS)],
                        agg_s.at[pl.ds(N_NODES, PAD_ROWS)])

    plsc.subcore_barrier()

    def step(j, carry):
        pltpu.sync_copy(src_hbm.at[wid, j], src_v)
        pltpu.sync_copy(dst_hbm.at[wid, j], dst_v)
        pltpu.async_copy(x_hbm.at[src_v], rows_v, sem).wait()
        pltpu.sync_copy(rows_v, agg_s.at[dst_v], add=True)
        return carry

    lax.fori_loop(0, K, step, 0)

    plsc.subcore_barrier()

    pltpu.sync_copy(agg_s.at[pl.ds(s * ROWS_PER_TILE, ROWS_PER_TILE)],
                    parts_hbm.at[c, pl.ds(s * ROWS_PER_TILE, ROWS_PER_TILE)])


def _tc_body(x_ref, p0_ref, p1_ref, wr_ref, wn_ref, b_ref, o_ref):
    agg = p0_ref[...] + p1_ref[...]
    acc = jnp.dot(x_ref[...], wr_ref[...], preferred_element_type=jnp.float32)
    acc = acc + jnp.dot(agg, wn_ref[...], preferred_element_type=jnp.float32)
    o_ref[...] = jnp.maximum(acc + b_ref[...], 0.0)


_ROW_BLK = 1000

_tc_finish = pl.pallas_call(
    _tc_body,
    grid=(N_NODES // _ROW_BLK,),
    in_specs=[
        pl.BlockSpec((_ROW_BLK, F), lambda i: (i, 0)),
        pl.BlockSpec((_ROW_BLK, F), lambda i: (i, 0)),
        pl.BlockSpec((_ROW_BLK, F), lambda i: (i, 0)),
        pl.BlockSpec((F, F), lambda i: (0, 0)),
        pl.BlockSpec((F, F), lambda i: (0, 0)),
        pl.BlockSpec((1, F), lambda i: (0, 0)),
    ],
    out_specs=pl.BlockSpec((_ROW_BLK, F), lambda i: (i, 0)),
    out_shape=jax.ShapeDtypeStruct((N_NODES, F), jnp.float32),
)


@jax.jit
def kernel(x, edge_index, W_root, W_nbr, b):
    ei = edge_index.astype(jnp.int32)
    pad = E_PAD - N_EDGES
    src = jnp.concatenate([ei[0], jnp.zeros((pad,), jnp.int32)])
    dst = jnp.concatenate([ei[1], jnp.full((pad,), N_NODES, jnp.int32)])
    src_r = src.reshape(NW, K, CHUNK)
    dst_r = dst.reshape(NW, K, CHUNK)
    zeros = jnp.zeros((ROWS_PER_TILE, F), jnp.float32)
    parts = _sc_aggregate(x, src_r, dst_r, zeros)
    return _tc_finish(x, parts[0], parts[1], W_root, W_nbr,
                      b.reshape(1, F))


# speedup vs baseline: 4.6667x; 4.6667x over previous
import functools

import jax
import jax.numpy as jnp
from jax import lax
from jax.experimental import pallas as pl
from jax.experimental.pallas import tpu as pltpu
from jax.experimental.pallas import tpu_sc as plsc

N_NODES = 10000
N_EDGES = 320000
F = 128

NC = 2
NS = 16
NW = NC * NS

CHUNK = 128
K = -(-N_EDGES // (NW * CHUNK))
E_PAD = NW * CHUNK * K
PAD_ROWS = 8

ROWS_MAIN = 624
LAST_START = (NS - 1) * ROWS_MAIN
LAST_ROWS = N_NODES - LAST_START
ZROWS = LAST_ROWS + PAD_ROWS

_sc_mesh = plsc.VectorSubcoreMesh(core_axis_name="c", subcore_axis_name="s")


@functools.partial(
    pl.kernel,
    out_type=jax.ShapeDtypeStruct((NC, N_NODES, F), jnp.float32),
    mesh=_sc_mesh,
    scratch_types=[
        pltpu.VMEM_SHARED((N_NODES + PAD_ROWS, F), jnp.float32),
        pltpu.VMEM((CHUNK,), jnp.int32),
        pltpu.VMEM((CHUNK,), jnp.int32),
        pltpu.VMEM((CHUNK, F), jnp.float32),
        pltpu.SemaphoreType.DMA,
    ],
)
def _sc_aggregate(x_hbm, src_hbm, dst_hbm, z_hbm, parts_hbm,
                  agg_s, src_v, dst_v, rows_v, sem):
    c = lax.axis_index("c")
    s = lax.axis_index("s")
    wid = c * NS + s

    start = pl.multiple_of(s * ROWS_MAIN, 8)

    @pl.when(s < NS - 1)
    def _():
        pltpu.sync_copy(z_hbm.at[pl.ds(0, ROWS_MAIN)],
                        agg_s.at[pl.ds(start, ROWS_MAIN)])

    @pl.when(s == NS - 1)
    def _():
        pltpu.sync_copy(z_hbm.at[pl.ds(0, ZROWS)],
                        agg_s.at[pl.ds(LAST_START, ZROWS)])

    plsc.subcore_barrier()

    def step(j, carry):
        pltpu.sync_copy(src_hbm.at[wid, j], src_v)
        pltpu.sync_copy(dst_hbm.at[wid, j], dst_v)
        pltpu.async_copy(x_hbm.at[src_v], rows_v, sem).wait()
        pltpu.sync_copy(rows_v, agg_s.at[dst_v], add=True)
        return carry

    lax.fori_loop(0, K, step, 0)

    plsc.subcore_barrier()

    @pl.when(s < NS - 1)
    def _():
        pltpu.sync_copy(agg_s.at[pl.ds(start, ROWS_MAIN)],
                        parts_hbm.at[c, pl.ds(start, ROWS_MAIN)])

    @pl.when(s == NS - 1)
    def _():
        pltpu.sync_copy(agg_s.at[pl.ds(LAST_START, LAST_ROWS)],
                        parts_hbm.at[c, pl.ds(LAST_START, LAST_ROWS)])


def _tc_body(x_ref, p0_ref, p1_ref, wr_ref, wn_ref, b_ref, o_ref):
    agg = p0_ref[...] + p1_ref[...]
    acc = jnp.dot(x_ref[...], wr_ref[...], preferred_element_type=jnp.float32)
    acc = acc + jnp.dot(agg, wn_ref[...], preferred_element_type=jnp.float32)
    o_ref[...] = jnp.maximum(acc + b_ref[...], 0.0)


_ROW_BLK = 1000

_tc_finish = pl.pallas_call(
    _tc_body,
    grid=(N_NODES // _ROW_BLK,),
    in_specs=[
        pl.BlockSpec((_ROW_BLK, F), lambda i: (i, 0)),
        pl.BlockSpec((_ROW_BLK, F), lambda i: (i, 0)),
        pl.BlockSpec((_ROW_BLK, F), lambda i: (i, 0)),
        pl.BlockSpec((F, F), lambda i: (0, 0)),
        pl.BlockSpec((F, F), lambda i: (0, 0)),
        pl.BlockSpec((1, F), lambda i: (0, 0)),
    ],
    out_specs=pl.BlockSpec((_ROW_BLK, F), lambda i: (i, 0)),
    out_shape=jax.ShapeDtypeStruct((N_NODES, F), jnp.float32),
)


@jax.jit
def kernel(x, edge_index, W_root, W_nbr, b):
    ei = edge_index.astype(jnp.int32)
    pad = E_PAD - N_EDGES
    src = jnp.concatenate([ei[0], jnp.zeros((pad,), jnp.int32)])
    dst = jnp.concatenate([ei[1], jnp.full((pad,), N_NODES, jnp.int32)])
    src_r = src.reshape(NW, K, CHUNK)
    dst_r = dst.reshape(NW, K, CHUNK)
    zeros = jnp.zeros((ZROWS, F), jnp.float32)
    parts = _sc_aggregate(x, src_r, dst_r, zeros)
    return _tc_finish(x, parts[0], parts[1], W_root, W_nbr,
                      b.reshape(1, F))
